# baseline (speedup 1.0000x reference)
import functools

import jax
import jax.numpy as jnp
from jax import lax
from jax.experimental import pallas as pl
from jax.experimental.pallas import tpu as pltpu
from jax.experimental.pallas import tpu_sc as plsc

B_MAX = 10.0
NUM_REL = 1_000_000

NC, NS, L = 2, 16, 16
NW = NC * NS

ROWS, COLS = 16384, 200
ROWS_PER_W = ROWS // NW
RPC = 64
NCHUNK = ROWS_PER_W // RPC
CHUNK = RPC * COLS

TBL_WIN = 64_000
WIN_CHUNKS = TBL_WIN // CHUNK
LAST_WIN = NUM_REL - TBL_WIN


def _body(ids_hbm, w_hbm, out_hbm, tbl_s, idx_v, val_v, wrow_v, sem):
    cid = lax.axis_index("c")
    sid = lax.axis_index("s")

    t0 = jnp.minimum(sid * TBL_WIN, LAST_WIN)

    def s1(c, carry):
        off = t0 + c * CHUNK
        pltpu.sync_copy(w_hbm.at[pl.ds(off, CHUNK)], wrow_v)

        def sig(i, inner):
            x = wrow_v[pl.ds(i * L, L)]
            wrow_v[pl.ds(i * L, L)] = B_MAX / (1.0 + jnp.exp(-x))
            return inner

        lax.fori_loop(0, CHUNK // L, sig, 0)
        pltpu.sync_copy(wrow_v, tbl_s.at[pl.ds(off, CHUNK)])
        return carry

    lax.fori_loop(0, WIN_CHUNKS, s1, 0)
    plsc.subcore_barrier()

    wid = sid * NC + cid
    base = wid * ROWS_PER_W

    def s2(c, carry):
        r0 = base + c * RPC
        pltpu.sync_copy(ids_hbm.at[pl.ds(r0, RPC), :], idx_v)

        def fire(j, inner):
            pltpu.async_copy(tbl_s.at[idx_v.at[j]], val_v.at[j], sem)
            return inner

        lax.fori_loop(0, RPC, fire, 0)
        pltpu.make_async_copy(ids_hbm.at[pl.ds(r0, RPC), :], val_v, sem).wait()
        pltpu.sync_copy(val_v, out_hbm.at[pl.ds(r0, RPC), :])
        return carry

    lax.fori_loop(0, NCHUNK, s2, 0)


_mesh = plsc.VectorSubcoreMesh(core_axis_name="c", subcore_axis_name="s")

_sc_call = functools.partial(
    pl.kernel,
    out_type=jax.ShapeDtypeStruct((ROWS, COLS), jnp.float32),
    mesh=_mesh,
    scratch_types=[
        pltpu.VMEM_SHARED((NUM_REL,), jnp.float32),
        pltpu.VMEM((RPC, COLS), jnp.int32),
        pltpu.VMEM((RPC, COLS), jnp.float32),
        pltpu.VMEM((CHUNK,), jnp.float32),
        pltpu.SemaphoreType.DMA,
    ],
    compiler_params=pltpu.CompilerParams(use_tc_tiling_on_sc=False),
)(_body)


def kernel(r_ids, w):
    return _sc_call(r_ids.astype(jnp.int32), w.reshape(-1))


# speedup vs baseline: 1.4891x; 1.4891x over previous
import functools

import jax
import jax.numpy as jnp
from jax import lax
from jax.experimental import pallas as pl
from jax.experimental.pallas import tpu as pltpu
from jax.experimental.pallas import tpu_sc as plsc

B_MAX = 10.0
NUM_REL = 1_000_000

NC, NS, L = 2, 16, 16
NW = NC * NS

ROWS, COLS = 16384, 200
ROWS_PER_W = ROWS // NW
RPC = 64
NCHUNK = ROWS_PER_W // RPC
CHUNK = RPC * COLS

TBL_WIN = 64_000
LAST_WIN = NUM_REL - TBL_WIN

NFULL = COLS // L
TAIL_OFF = COLS - L


def _body(ids_hbm, w_hbm, out_hbm, tbl_s, idx0, idx1, val0, val1,
          sem_i, sem_g, sem_o):
    cid = lax.axis_index("c")
    sid = lax.axis_index("s")
    idxb = (idx0, idx1)
    valb = (val0, val1)

    wid = sid * NC + cid
    base = wid * ROWS_PER_W

    def ids_start(c):
        pltpu.async_copy(ids_hbm.at[pl.ds(base + c * RPC, RPC), :],
                         idxb[c % 2], sem_i)

    def wait_ids(c):
        pltpu.make_async_copy(ids_hbm.at[pl.ds(0, RPC), :], idxb[c % 2],
                              sem_i).wait()

    def wait_out(c):
        pltpu.make_async_copy(valb[c % 2],
                              out_hbm.at[pl.ds(0, RPC), :], sem_o).wait()

    ids_start(0)
    ids_start(1)

    t0 = jnp.minimum(sid * TBL_WIN, LAST_WIN)
    pltpu.sync_copy(w_hbm.at[pl.ds(t0, TBL_WIN)], tbl_s.at[pl.ds(t0, TBL_WIN)])
    plsc.subcore_barrier()

    lanes = lax.iota(jnp.int32, L)

    def gathers(c):
        b = valb[c % 2]
        ib = idxb[c % 2]

        def fire(j, inner):
            pltpu.async_copy(tbl_s.at[ib.at[j]], b.at[j], sem_g)
            return inner

        lax.fori_loop(0, RPC, fire, 0)

    def drain_gathers(c):
        pltpu.make_async_copy(ids_hbm.at[pl.ds(0, RPC), :], valb[c % 2],
                              sem_g).wait()

    def sigmoid_and_out(c):
        buf = valb[c % 2]

        def row(j, inner):
            for s in range(NFULL):
                x = buf[j, pl.ds(s * L, L)]
                buf[j, pl.ds(s * L, L)] = B_MAX / (1.0 + jnp.exp(-x))
            x = buf[j, pl.ds(TAIL_OFF, L)]
            y = B_MAX / (1.0 + jnp.exp(-x))
            buf[j, pl.ds(TAIL_OFF, L)] = jnp.where(lanes >= L // 2, y, x)
            return inner

        lax.fori_loop(0, RPC, row, 0)
        pltpu.async_copy(buf, out_hbm.at[pl.ds(base + c * RPC, RPC), :], sem_o)

    for c in range(NCHUNK + 1):
        if c < NCHUNK:
            if c >= 2:
                wait_out(c - 2)
            wait_ids(c)
            gathers(c)
            if 1 <= c < NCHUNK - 1:
                ids_start(c + 1)
        if c >= 1:
            sigmoid_and_out(c - 1)
        if c < NCHUNK:
            drain_gathers(c)

    wait_out(NCHUNK - 2)
    wait_out(NCHUNK - 1)


_mesh = plsc.VectorSubcoreMesh(core_axis_name="c", subcore_axis_name="s")

_sc_call = functools.partial(
    pl.kernel,
    out_type=jax.ShapeDtypeStruct((ROWS, COLS), jnp.float32),
    mesh=_mesh,
    scratch_types=[
        pltpu.VMEM_SHARED((NUM_REL,), jnp.float32),
        pltpu.VMEM((RPC, COLS), jnp.int32),
        pltpu.VMEM((RPC, COLS), jnp.int32),
        pltpu.VMEM((RPC, COLS), jnp.float32),
        pltpu.VMEM((RPC, COLS), jnp.float32),
        pltpu.SemaphoreType.DMA,
        pltpu.SemaphoreType.DMA,
        pltpu.SemaphoreType.DMA,
    ],
    compiler_params=pltpu.CompilerParams(use_tc_tiling_on_sc=False),
)(_body)


def kernel(r_ids, w):
    return _sc_call(r_ids.astype(jnp.int32), w.reshape(-1))


# speedup vs baseline: 1.5170x; 1.0187x over previous
import functools

import jax
import jax.numpy as jnp
from jax import lax
from jax.experimental import pallas as pl
from jax.experimental.pallas import tpu as pltpu
from jax.experimental.pallas import tpu_sc as plsc

B_MAX = 10.0
NUM_REL = 1_000_000

NC, NS, L = 2, 16, 16
NW = NC * NS

ROWS, COLS = 16384, 200
TOTAL = ROWS * COLS
PER_W = TOTAL // NW
CHUNK = 12_800
NCHUNK = PER_W // CHUNK
NSTREAM = 8
SUB = CHUNK // NSTREAM
UNROLL = 4

TBL_WIN = 64_000
LAST_WIN = NUM_REL - TBL_WIN


def _body(ids_hbm, w_hbm, out_hbm, tbl_s, idx0, idx1, val0, val1,
          sem_i, sem_g, sem_o):
    cid = lax.axis_index("c")
    sid = lax.axis_index("s")
    idxb = (idx0, idx1)
    valb = (val0, val1)

    wid = sid * NC + cid
    base = wid * PER_W

    def ids_start(c):
        pltpu.async_copy(ids_hbm.at[pl.ds(base + c * CHUNK, CHUNK)],
                         idxb[c % 2], sem_i)

    def wait_ids(c):
        pltpu.make_async_copy(ids_hbm.at[pl.ds(0, CHUNK)], idxb[c % 2],
                              sem_i).wait()

    def wait_out(c):
        pltpu.make_async_copy(valb[c % 2], out_hbm.at[pl.ds(0, CHUNK)],
                              sem_o).wait()

    ids_start(0)
    ids_start(1)

    t0 = jnp.minimum(sid * TBL_WIN, LAST_WIN)
    pltpu.sync_copy(w_hbm.at[pl.ds(t0, TBL_WIN)], tbl_s.at[pl.ds(t0, TBL_WIN)])
    plsc.subcore_barrier()

    def gathers(c):
        ib, vb = idxb[c % 2], valb[c % 2]
        for k in range(NSTREAM):
            pltpu.async_copy(tbl_s.at[ib.at[pl.ds(k * SUB, SUB)]],
                             vb.at[pl.ds(k * SUB, SUB)], sem_g)

    def drain_gathers(c):
        pltpu.make_async_copy(ids_hbm.at[pl.ds(0, CHUNK)], valb[c % 2],
                              sem_g).wait()

    def sigmoid_and_out(c):
        buf = valb[c % 2]

        def blk(i, carry):
            for u in range(UNROLL):
                o = (i * UNROLL + u) * L
                x = buf[pl.ds(o, L)]
                buf[pl.ds(o, L)] = B_MAX / (1.0 + jnp.exp(-x))
            return carry

        lax.fori_loop(0, CHUNK // (L * UNROLL), blk, 0)
        pltpu.async_copy(buf, out_hbm.at[pl.ds(base + c * CHUNK, CHUNK)],
                         sem_o)

    for c in range(NCHUNK + 1):
        if c < NCHUNK:
            if c >= 2:
                wait_out(c - 2)
            wait_ids(c)
            gathers(c)
            if 1 <= c < NCHUNK - 1:
                ids_start(c + 1)
        if c >= 1:
            sigmoid_and_out(c - 1)
        if c < NCHUNK:
            drain_gathers(c)

    wait_out(NCHUNK - 2)
    wait_out(NCHUNK - 1)


_mesh = plsc.VectorSubcoreMesh(core_axis_name="c", subcore_axis_name="s")

_sc_call = functools.partial(
    pl.kernel,
    out_type=jax.ShapeDtypeStruct((TOTAL,), jnp.float32),
    mesh=_mesh,
    scratch_types=[
        pltpu.VMEM_SHARED((NUM_REL,), jnp.float32),
        pltpu.VMEM((CHUNK,), jnp.int32),
        pltpu.VMEM((CHUNK,), jnp.int32),
        pltpu.VMEM((CHUNK,), jnp.float32),
        pltpu.VMEM((CHUNK,), jnp.float32),
        pltpu.SemaphoreType.DMA,
        pltpu.SemaphoreType.DMA,
        pltpu.SemaphoreType.DMA,
    ],
    compiler_params=pltpu.CompilerParams(use_tc_tiling_on_sc=False),
)(_body)


def kernel(r_ids, w):
    out = _sc_call(r_ids.reshape(-1).astype(jnp.int32), w.reshape(-1))
    return out.reshape(ROWS, COLS)
